# baseline (speedup 1.0000x reference)
import functools
import math

import jax
import jax.numpy as jnp
from jax import lax
from jax.experimental import pallas as pl
from jax.experimental.pallas import tpu as pltpu
from jax.experimental.pallas import tpu_sc as plsc

_NTOKEN = 100000
_NINP = 1024
_SEQ = 4096
_BATCH = 4

_P = 4
_SEQ_P = _SEQ // _P
_NROWS_P = _SEQ_P * _BATCH

_NC = 2
_NS = 16
_NW = _NC * _NS
_ROWS_PER_W = _NROWS_P // _NW
_CHUNK = 64
_NCHUNK = _ROWS_PER_W // _CHUNK


def _sc_gather_body(table_hbm, idx_hbm, out_hbm, idx_v, rows_v, sem):
    wid = lax.axis_index("s") * _NC + lax.axis_index("c")
    base = wid * _ROWS_PER_W
    for c in range(_NCHUNK):
        off = base + c * _CHUNK
        pltpu.sync_copy(idx_hbm.at[pl.ds(off, _CHUNK)], idx_v)
        pltpu.async_copy(table_hbm.at[idx_v], rows_v, sem).wait()
        pltpu.sync_copy(rows_v, out_hbm.at[pl.ds(off, _CHUNK)])


@functools.cache
def _sc_gather():
    return pl.kernel(
        _sc_gather_body,
        mesh=plsc.VectorSubcoreMesh(core_axis_name="c", subcore_axis_name="s"),
        out_type=jax.ShapeDtypeStruct((_NROWS_P, _NINP), jnp.float32),
        scratch_types=[
            pltpu.VMEM((_CHUNK,), jnp.int32),
            pltpu.VMEM((_CHUNK, _NINP), jnp.float32),
            pltpu.SemaphoreType.DMA,
        ],
    )


_S_BLK = 256


def _logsoftmax_body(g_ref, pe_ref, _, o_ref):
    pe3 = pe_ref[...]
    pe_exp = jnp.broadcast_to(
        pe3, (_S_BLK, _BATCH, _NINP)
    ).reshape(_S_BLK * _BATCH, _NINP)
    y = g_ref[...] * math.sqrt(_NINP) + pe_exp
    m = jnp.max(y, axis=-1, keepdims=True)
    e = jnp.exp(y - m)
    s = jnp.sum(e, axis=-1, keepdims=True)
    out2 = y - m - jnp.log(s)
    o_ref[...] = out2.reshape(_S_BLK, _BATCH, _NINP)


def _tc_logsoftmax_chunk(g2, pe, prev, p):
    blk_off = p * (_SEQ_P // _S_BLK)
    return pl.pallas_call(
        _logsoftmax_body,
        grid=(_SEQ_P // _S_BLK,),
        in_specs=[
            pl.BlockSpec((_S_BLK * _BATCH, _NINP), lambda i: (i, 0)),
            pl.BlockSpec((_S_BLK, 1, _NINP),
                         lambda i, _o=blk_off: (_o + i, 0, 0)),
            pl.BlockSpec(memory_space=pl.ANY),
        ],
        out_specs=pl.BlockSpec((_S_BLK, _BATCH, _NINP),
                               lambda i, _o=blk_off: (_o + i, 0, 0)),
        out_shape=jax.ShapeDtypeStruct((_SEQ, _BATCH, _NINP), jnp.float32),
        input_output_aliases={2: 0},
    )(g2, pe, prev)


def kernel(src, emb_weight, pe):
    idx = src.reshape(-1).astype(jnp.int32)
    gathered = [
        _sc_gather()(emb_weight, lax.slice(idx, (p * _NROWS_P,),
                                           ((p + 1) * _NROWS_P,)))
        for p in range(_P)
    ]
    out = pl.pallas_call(
        lambda g_ref, pe_ref, o_ref: _logsoftmax_body(
            g_ref, pe_ref, None, o_ref),
        grid=(_SEQ_P // _S_BLK,),
        in_specs=[
            pl.BlockSpec((_S_BLK * _BATCH, _NINP), lambda i: (i, 0)),
            pl.BlockSpec((_S_BLK, 1, _NINP), lambda i: (i, 0, 0)),
        ],
        out_specs=pl.BlockSpec((_S_BLK, _BATCH, _NINP),
                               lambda i: (i, 0, 0)),
        out_shape=jax.ShapeDtypeStruct((_SEQ, _BATCH, _NINP), jnp.float32),
    )(gathered[0], pe)
    for p in range(1, _P):
        out = _tc_logsoftmax_chunk(gathered[p], pe, out, p)
    return out


# speedup vs baseline: 1.0141x; 1.0141x over previous
import functools
import math

import jax
import jax.numpy as jnp
from jax import lax
from jax.experimental import pallas as pl
from jax.experimental.pallas import tpu as pltpu
from jax.experimental.pallas import tpu_sc as plsc

_NTOKEN = 100000
_NINP = 1024
_SEQ = 4096
_BATCH = 4

_P = 4
_SEQ_P = _SEQ // _P
_NROWS_P = _SEQ_P * _BATCH

_NC = 2
_NS = 16
_NW = _NC * _NS
_ROWS_PER_W = _NROWS_P // _NW
_CHUNK = 32
_NCHUNK = _ROWS_PER_W // _CHUNK


def _sc_gather_body(table_hbm, idx_hbm, out_hbm,
                    idx_v, rows0, rows1, gsem0, gsem1):
    wid = lax.axis_index("s") * _NC + lax.axis_index("c")
    base = wid * _ROWS_PER_W
    bufs = (rows0, rows1)
    sems = (gsem0, gsem1)
    pltpu.sync_copy(idx_hbm.at[pl.ds(base, _ROWS_PER_W)], idx_v)

    def _gather(c):
        return pltpu.async_copy(
            table_hbm.at[idx_v.at[pl.ds(c * _CHUNK, _CHUNK)]],
            bufs[c % 2], sems[c % 2])

    copies = {0: _gather(0)}
    for c in range(_NCHUNK):
        if c + 1 < _NCHUNK:
            copies[c + 1] = _gather(c + 1)
        copies[c].wait()
        pltpu.sync_copy(bufs[c % 2],
                        out_hbm.at[pl.ds(base + c * _CHUNK, _CHUNK)])


@functools.cache
def _sc_gather():
    return pl.kernel(
        _sc_gather_body,
        mesh=plsc.VectorSubcoreMesh(core_axis_name="c", subcore_axis_name="s"),
        out_type=jax.ShapeDtypeStruct((_NROWS_P, _NINP), jnp.float32),
        scratch_types=[
            pltpu.VMEM((_ROWS_PER_W,), jnp.int32),
            pltpu.VMEM((_CHUNK, _NINP), jnp.float32),
            pltpu.VMEM((_CHUNK, _NINP), jnp.float32),
            pltpu.SemaphoreType.DMA,
            pltpu.SemaphoreType.DMA,
        ],
    )


_S_BLK = 256


def _logsoftmax_body(g_ref, pe_ref, _, o_ref):
    pe3 = pe_ref[...]
    pe_exp = jnp.broadcast_to(
        pe3, (_S_BLK, _BATCH, _NINP)
    ).reshape(_S_BLK * _BATCH, _NINP)
    y = g_ref[...] * math.sqrt(_NINP) + pe_exp
    m = jnp.max(y, axis=-1, keepdims=True)
    e = jnp.exp(y - m)
    s = jnp.sum(e, axis=-1, keepdims=True)
    out2 = y - m - jnp.log(s)
    o_ref[...] = out2.reshape(_S_BLK, _BATCH, _NINP)


def _tc_logsoftmax_chunk(g2, pe, prev, p):
    blk_off = p * (_SEQ_P // _S_BLK)
    return pl.pallas_call(
        _logsoftmax_body,
        grid=(_SEQ_P // _S_BLK,),
        in_specs=[
            pl.BlockSpec((_S_BLK * _BATCH, _NINP), lambda i: (i, 0)),
            pl.BlockSpec((_S_BLK, 1, _NINP),
                         lambda i, _o=blk_off: (_o + i, 0, 0)),
            pl.BlockSpec(memory_space=pl.ANY),
        ],
        out_specs=pl.BlockSpec((_S_BLK, _BATCH, _NINP),
                               lambda i, _o=blk_off: (_o + i, 0, 0)),
        out_shape=jax.ShapeDtypeStruct((_SEQ, _BATCH, _NINP), jnp.float32),
        input_output_aliases={2: 0},
    )(g2, pe, prev)


def kernel(src, emb_weight, pe):
    idx = src.reshape(-1).astype(jnp.int32)
    gathered = [
        _sc_gather()(emb_weight, lax.slice(idx, (p * _NROWS_P,),
                                           ((p + 1) * _NROWS_P,)))
        for p in range(_P)
    ]
    out = pl.pallas_call(
        lambda g_ref, pe_ref, o_ref: _logsoftmax_body(
            g_ref, pe_ref, None, o_ref),
        grid=(_SEQ_P // _S_BLK,),
        in_specs=[
            pl.BlockSpec((_S_BLK * _BATCH, _NINP), lambda i: (i, 0)),
            pl.BlockSpec((_S_BLK, 1, _NINP), lambda i: (i, 0, 0)),
        ],
        out_specs=pl.BlockSpec((_S_BLK, _BATCH, _NINP),
                               lambda i: (i, 0, 0)),
        out_shape=jax.ShapeDtypeStruct((_SEQ, _BATCH, _NINP), jnp.float32),
    )(gathered[0], pe)
    for p in range(1, _P):
        out = _tc_logsoftmax_chunk(gathered[p], pe, out, p)
    return out
